# Initial kernel scaffold; baseline (speedup 1.0000x reference)
import jax
import jax.numpy as jnp
from jax.experimental import pallas as pl


def kernel(x, edge_index, W, b):
    raise NotImplementedError("write your pallas kernel here")


# speedup vs baseline: 5.3596x; 5.3596x over previous
import functools

import jax
import jax.numpy as jnp
from jax import lax
from jax.experimental import pallas as pl
from jax.experimental.pallas import tpu as pltpu
from jax.experimental.pallas import tpu_sc as plsc

N_NODES = 10000
N_EDGES = 320000
D = 128
H = D // 2

NC = 2
NS = 16

CHUNK = 80
CHUNKS_PER_TILE = N_EDGES // (NS * CHUNK)

NP = 10240
ROWS_PER_TILE = NP // NS

BM = 400


def _mm_body(x_ref, w1_ref, w2_ref, b_ref, t_ref):
    xb = x_ref[...]
    m1 = jnp.dot(xb, w1_ref[...], preferred_element_type=jnp.float32) + b_ref[...]
    m2 = jnp.dot(xb, w2_ref[...], preferred_element_type=jnp.float32)
    t_ref[0] = m1[:, :H]
    t_ref[1] = m1[:, H:]
    t_ref[2] = m2[:, :H]
    t_ref[3] = m2[:, H:]


def _precompute_table(x, w1, w2, b2d):
    return pl.pallas_call(
        _mm_body,
        grid=(N_NODES // BM,),
        in_specs=[
            pl.BlockSpec((BM, D), lambda i: (i, 0)),
            pl.BlockSpec((D, D), lambda i: (0, 0)),
            pl.BlockSpec((D, D), lambda i: (0, 0)),
            pl.BlockSpec((1, D), lambda i: (0, 0)),
        ],
        out_specs=pl.BlockSpec((4, BM, H), lambda i: (0, i, 0)),
        out_shape=jax.ShapeDtypeStruct((4, N_NODES, H), jnp.float32),
    )(x, w1, w2, b2d)


def _combine_body(p_ref, o_ref):
    o_ref[:, :H] = p_ref[0]
    o_ref[:, H:] = p_ref[1]


def _combine(partials):
    return pl.pallas_call(
        _combine_body,
        grid=(N_NODES // BM,),
        in_specs=[pl.BlockSpec((NC, BM, H), lambda i: (0, i, 0))],
        out_specs=pl.BlockSpec((BM, D), lambda i: (i, 0)),
        out_shape=jax.ShapeDtypeStruct((N_NODES, D), jnp.float32),
    )(partials)


@functools.partial(
    pl.kernel,
    out_type=jax.ShapeDtypeStruct((NC, NP, H), jnp.float32),
    mesh=plsc.VectorSubcoreMesh(core_axis_name="c", subcore_axis_name="s"),
    scratch_types=[
        pltpu.VMEM((CHUNKS_PER_TILE, CHUNK), jnp.int32),
        pltpu.VMEM((CHUNKS_PER_TILE, CHUNK), jnp.int32),
        pltpu.VMEM((CHUNKS_PER_TILE, CHUNK), jnp.int32),
        pltpu.VMEM((CHUNK, H), jnp.float32),
        pltpu.VMEM((CHUNK, H), jnp.float32),
        pltpu.VMEM_SHARED((NP, H), jnp.float32),
        pltpu.SemaphoreType.DMA,
        pltpu.SemaphoreType.DMA,
    ],
    compiler_params=pltpu.CompilerParams(use_tc_tiling_on_sc=False),
)
def _sc_edges(t_hbm, srcg_hbm, dstg_hbm, dsts_hbm, out_hbm,
              sidx, didx, kidx, ra, rb, accum, sem_a, sem_b):
    c = lax.axis_index("c")
    s = lax.axis_index("s")

    zero = jnp.zeros((16,), jnp.float32)

    @pl.loop(0, CHUNK)
    def _zero_rows(e):
        for j in range(H // 16):
            ra[e, pl.ds(j * 16, 16)] = zero

    row0 = s * ROWS_PER_TILE
    for k in range(ROWS_PER_TILE // CHUNK):
        pltpu.sync_copy(ra, accum.at[pl.ds(row0 + k * CHUNK, CHUNK)])

    pltpu.sync_copy(srcg_hbm.at[c, s], sidx)
    pltpu.sync_copy(dstg_hbm.at[c, s], didx)
    pltpu.sync_copy(dsts_hbm.at[s], kidx)

    plsc.subcore_barrier()

    @pl.loop(0, CHUNKS_PER_TILE)
    def _chunk(g):
        cp_a = pltpu.async_copy(t_hbm.at[sidx.at[g]], ra, sem_a)
        cp_b = pltpu.async_copy(t_hbm.at[didx.at[g]], rb, sem_b)
        cp_a.wait()
        cp_b.wait()

        @pl.loop(0, CHUNK)
        def _row(e):
            for j in range(H // 16):
                sl = pl.ds(j * 16, 16)
                ra[e, sl] = jnp.maximum(ra[e, sl] + rb[e, sl], 0.0)

        pltpu.sync_copy(ra, accum.at[kidx.at[g]], add=True)

    plsc.subcore_barrier()
    pltpu.sync_copy(
        accum.at[pl.ds(row0, ROWS_PER_TILE)],
        out_hbm.at[c, pl.ds(row0, ROWS_PER_TILE)],
    )


def kernel(x, edge_index, W, b):
    w1 = W[:D]
    w2 = W[D:]
    b2d = b.reshape(1, D)
    table = _precompute_table(x, w1, w2, b2d).reshape(4 * N_NODES, H)
    src = edge_index[0].reshape(NS, CHUNKS_PER_TILE, CHUNK)
    dst = edge_index[1].reshape(NS, CHUNKS_PER_TILE, CHUNK)
    srcg = jnp.stack([src, src + N_NODES])
    dstg = jnp.stack([dst + 2 * N_NODES, dst + 3 * N_NODES])
    partials = _sc_edges(table, srcg, dstg, dst)
    return _combine(partials)
